# baseline (speedup 1.0000x reference)
import functools

import jax
import jax.numpy as jnp
from jax.experimental import pallas as pl
from jax.experimental.pallas import tpu as pltpu

_C = ((0.5, 0.25), (0.75, -0.25))
_BLOCK_ROWS = 1600


def _body(in0_ref, in1_ref, out_ref):
    r = _BLOCK_ROWS
    in0 = in0_ref[...].reshape(r, 32)
    in1 = in1_ref[...]

    k_t = jax.lax.broadcasted_iota(jnp.int32, (32, 256), 0)
    l_t = jax.lax.broadcasted_iota(jnp.int32, (32, 256), 1)
    same_t = (l_t // 128) == (k_t // 16)
    same_u = ((l_t % 128) // 8) == (k_t % 16)
    scale = jnp.where(k_t // 16 == 0, _C[0][0], _C[1][0]).astype(jnp.float32)
    B = jnp.where(same_t & same_u, scale, 0.0)

    r_i = jax.lax.broadcasted_iota(jnp.int32, (256, 16), 0)
    c_i = jax.lax.broadcasted_iota(jnp.int32, (256, 16), 1)
    S = jnp.where(
        ((r_i // 128) == (c_i // 8)) & ((r_i % 8) == (c_i % 8)), 1.0, 0.0
    ).astype(jnp.float32)

    in1a = in1[:, :128]
    in1b = in1[:, 128:]
    m0 = in1a + (_C[0][1] / _C[0][0]) * in1b
    m1 = in1a + (_C[1][1] / _C[1][0]) * in1b
    m = jnp.concatenate([m0, m1], axis=1)

    w = jax.lax.dot(in0, B, precision=jax.lax.Precision.DEFAULT)
    out = jax.lax.dot(w * m, S, precision=jax.lax.Precision.DEFAULT)

    out3 = out.reshape(r // 8, 8, 16)
    out_ref[...] = jnp.concatenate([out3[:, q, :] for q in range(8)], axis=1)


@jax.jit
def kernel(in0, in1):
    n = in0.shape[0]
    r = _BLOCK_ROWS
    grid = (n // r,)
    in0p = in0.reshape(n // 4, 4, 32)
    outp = pl.pallas_call(
        _body,
        grid=grid,
        in_specs=[
            pl.BlockSpec((r // 4, 4, 32), lambda i: (i, 0, 0)),
            pl.BlockSpec((r, 256), lambda i: (i, 0)),
        ],
        out_specs=pl.BlockSpec((r // 8, 128), lambda i: (i, 0)),
        out_shape=jax.ShapeDtypeStruct((n // 8, 128), in0.dtype),
        compiler_params=pltpu.CompilerParams(
            dimension_semantics=("arbitrary",),
        ),
    )(in0p, in1)
    return outp.reshape(n, 16)


# speedup vs baseline: 2.2948x; 2.2948x over previous
import functools

import jax
import jax.numpy as jnp
from jax.experimental import pallas as pl
from jax.experimental.pallas import tpu as pltpu

_C = ((0.5, 0.25), (0.75, -0.25))
_BLOCK_ROWS = 2048


def _body(in0t_ref, in1_ref, outt_ref):
    r = _BLOCK_ROWS
    in0 = in0t_ref[...].T
    in1 = in1_ref[...]

    k_t = jax.lax.broadcasted_iota(jnp.int32, (32, 256), 0)
    l_t = jax.lax.broadcasted_iota(jnp.int32, (32, 256), 1)
    same_t = (l_t // 128) == (k_t // 16)
    same_u = ((l_t % 128) // 8) == (k_t % 16)
    scale = jnp.where(k_t // 16 == 0, _C[0][0], _C[1][0]).astype(jnp.float32)
    B = jnp.where(same_t & same_u, scale, 0.0)

    r_i = jax.lax.broadcasted_iota(jnp.int32, (256, 16), 0)
    c_i = jax.lax.broadcasted_iota(jnp.int32, (256, 16), 1)
    S = jnp.where(
        ((r_i // 128) == (c_i // 8)) & ((r_i % 8) == (c_i % 8)), 1.0, 0.0
    ).astype(jnp.float32)

    in1a = in1[:, :128]
    in1b = in1[:, 128:]
    m0 = in1a + (_C[0][1] / _C[0][0]) * in1b
    m1 = in1a + (_C[1][1] / _C[1][0]) * in1b
    m = jnp.concatenate([m0, m1], axis=1)

    w = jax.lax.dot(in0, B, precision=jax.lax.Precision.DEFAULT)
    out = jax.lax.dot(w * m, S, precision=jax.lax.Precision.DEFAULT)
    outt_ref[...] = out.T


@jax.jit
def kernel(in0, in1):
    n = in0.shape[0]
    r = _BLOCK_ROWS
    grid = (pl.cdiv(n, r),)
    in0t = in0.T
    outt = pl.pallas_call(
        _body,
        grid=grid,
        in_specs=[
            pl.BlockSpec((32, r), lambda i: (0, i)),
            pl.BlockSpec((r, 256), lambda i: (i, 0)),
        ],
        out_specs=pl.BlockSpec((16, r), lambda i: (0, i)),
        out_shape=jax.ShapeDtypeStruct((16, n), in0.dtype),
        compiler_params=pltpu.CompilerParams(
            dimension_semantics=("arbitrary",),
        ),
    )(in0t, in1)
    return outt.T


# speedup vs baseline: 3.0660x; 1.3361x over previous
import functools

import jax
import jax.numpy as jnp
from jax.experimental import pallas as pl
from jax.experimental.pallas import tpu as pltpu

_C = ((0.5, 0.25), (0.75, -0.25))
_BLOCK_ROWS = 4096


def _body(in0t_ref, in1_ref, outt_ref):
    r = _BLOCK_ROWS
    in0 = in0t_ref[...].T
    in1 = in1_ref[...]

    k_t = jax.lax.broadcasted_iota(jnp.int32, (32, 256), 0)
    l_t = jax.lax.broadcasted_iota(jnp.int32, (32, 256), 1)
    same_t = (l_t // 128) == (k_t // 16)
    same_u = ((l_t % 128) // 8) == (k_t % 16)
    scale = jnp.where(k_t // 16 == 0, _C[0][0], _C[1][0]).astype(jnp.float32)
    B = jnp.where(same_t & same_u, scale, 0.0)

    r_i = jax.lax.broadcasted_iota(jnp.int32, (256, 16), 0)
    c_i = jax.lax.broadcasted_iota(jnp.int32, (256, 16), 1)
    S = jnp.where(
        ((r_i // 128) == (c_i // 8)) & ((r_i % 8) == (c_i % 8)), 1.0, 0.0
    ).astype(jnp.float32)

    in1a = in1[:, :128]
    in1b = in1[:, 128:]
    m0 = in1a + (_C[0][1] / _C[0][0]) * in1b
    m1 = in1a + (_C[1][1] / _C[1][0]) * in1b
    m = jnp.concatenate([m0, m1], axis=1)

    w = jax.lax.dot(in0, B, precision=jax.lax.Precision.DEFAULT)
    out = jax.lax.dot(w * m, S, precision=jax.lax.Precision.DEFAULT)
    outt_ref[...] = out.T


@jax.jit
def kernel(in0, in1):
    n = in0.shape[0]
    r = _BLOCK_ROWS
    grid = (pl.cdiv(n, r),)
    in0t = in0.T
    outt = pl.pallas_call(
        _body,
        grid=grid,
        in_specs=[
            pl.BlockSpec((32, r), lambda i: (0, i)),
            pl.BlockSpec((r, 256), lambda i: (i, 0)),
        ],
        out_specs=pl.BlockSpec((16, r), lambda i: (0, i)),
        out_shape=jax.ShapeDtypeStruct((16, n), in0.dtype),
        compiler_params=pltpu.CompilerParams(
            dimension_semantics=("arbitrary",),
        ),
    )(in0t, in1)
    return outt.T


# speedup vs baseline: 3.6508x; 1.1907x over previous
import functools

import jax
import jax.numpy as jnp
from jax.experimental import pallas as pl
from jax.experimental.pallas import tpu as pltpu

_C = ((0.5, 0.25), (0.75, -0.25))
_BLOCK_ROWS = 8192


def _body(in0t_ref, in1_ref, outt_ref):
    r = _BLOCK_ROWS
    in0 = in0t_ref[...].T
    in1 = in1_ref[...]

    k_t = jax.lax.broadcasted_iota(jnp.int32, (32, 256), 0)
    l_t = jax.lax.broadcasted_iota(jnp.int32, (32, 256), 1)
    same_t = (l_t // 128) == (k_t // 16)
    same_u = ((l_t % 128) // 8) == (k_t % 16)
    scale = jnp.where(k_t // 16 == 0, _C[0][0], _C[1][0]).astype(jnp.float32)
    B = jnp.where(same_t & same_u, scale, 0.0)

    r_i = jax.lax.broadcasted_iota(jnp.int32, (256, 16), 0)
    c_i = jax.lax.broadcasted_iota(jnp.int32, (256, 16), 1)
    S = jnp.where(
        ((r_i // 128) == (c_i // 8)) & ((r_i % 8) == (c_i % 8)), 1.0, 0.0
    ).astype(jnp.float32)

    in1a = in1[:, :128]
    in1b = in1[:, 128:]
    m0 = in1a + (_C[0][1] / _C[0][0]) * in1b
    m1 = in1a + (_C[1][1] / _C[1][0]) * in1b
    m = jnp.concatenate([m0, m1], axis=1)

    w = jax.lax.dot(in0, B, precision=jax.lax.Precision.DEFAULT)
    out = jax.lax.dot(w * m, S, precision=jax.lax.Precision.DEFAULT)
    outt_ref[...] = out.T


@jax.jit
def kernel(in0, in1):
    n = in0.shape[0]
    r = _BLOCK_ROWS
    grid = (pl.cdiv(n, r),)
    in0t = in0.T
    outt = pl.pallas_call(
        _body,
        grid=grid,
        in_specs=[
            pl.BlockSpec((32, r), lambda i: (0, i)),
            pl.BlockSpec((r, 256), lambda i: (i, 0)),
        ],
        out_specs=pl.BlockSpec((16, r), lambda i: (0, i)),
        out_shape=jax.ShapeDtypeStruct((16, n), in0.dtype),
        compiler_params=pltpu.CompilerParams(
            dimension_semantics=("arbitrary",),
        ),
    )(in0t, in1)
    return outt.T


# speedup vs baseline: 3.8100x; 1.0436x over previous
import functools

import jax
import jax.numpy as jnp
from jax.experimental import pallas as pl
from jax.experimental.pallas import tpu as pltpu

_C = ((0.5, 0.25), (0.75, -0.25))
_BLOCK_ROWS = 16384


def _body(in0t_ref, in1_ref, outt_ref):
    r = _BLOCK_ROWS
    in0 = in0t_ref[...].T
    in1 = in1_ref[...]

    k_t = jax.lax.broadcasted_iota(jnp.int32, (32, 256), 0)
    l_t = jax.lax.broadcasted_iota(jnp.int32, (32, 256), 1)
    same_t = (l_t // 128) == (k_t // 16)
    same_u = ((l_t % 128) // 8) == (k_t % 16)
    scale = jnp.where(k_t // 16 == 0, _C[0][0], _C[1][0]).astype(jnp.float32)
    B = jnp.where(same_t & same_u, scale, 0.0)

    r_i = jax.lax.broadcasted_iota(jnp.int32, (256, 16), 0)
    c_i = jax.lax.broadcasted_iota(jnp.int32, (256, 16), 1)
    S = jnp.where(
        ((r_i // 128) == (c_i // 8)) & ((r_i % 8) == (c_i % 8)), 1.0, 0.0
    ).astype(jnp.float32)

    in1a = in1[:, :128]
    in1b = in1[:, 128:]
    m0 = in1a + (_C[0][1] / _C[0][0]) * in1b
    m1 = in1a + (_C[1][1] / _C[1][0]) * in1b
    m = jnp.concatenate([m0, m1], axis=1)

    w = jax.lax.dot(in0, B, precision=jax.lax.Precision.DEFAULT)
    out = jax.lax.dot(w * m, S, precision=jax.lax.Precision.DEFAULT)
    outt_ref[...] = out.T


@jax.jit
def kernel(in0, in1):
    n = in0.shape[0]
    r = _BLOCK_ROWS
    grid = (pl.cdiv(n, r),)
    in0t = in0.T
    outt = pl.pallas_call(
        _body,
        grid=grid,
        in_specs=[
            pl.BlockSpec((32, r), lambda i: (0, i)),
            pl.BlockSpec((r, 256), lambda i: (i, 0)),
        ],
        out_specs=pl.BlockSpec((16, r), lambda i: (0, i)),
        out_shape=jax.ShapeDtypeStruct((16, n), in0.dtype),
        compiler_params=pltpu.CompilerParams(
            dimension_semantics=("arbitrary",),
        ),
    )(in0t, in1)
    return outt.T


# speedup vs baseline: 3.9303x; 1.0316x over previous
import functools

import jax
import jax.numpy as jnp
from jax.experimental import pallas as pl
from jax.experimental.pallas import tpu as pltpu

_C = ((0.5, 0.25), (0.75, -0.25))
_BLOCK_ROWS = 20480


def _body(in0t_ref, in1_ref, outt_ref):
    r = _BLOCK_ROWS
    in0 = in0t_ref[...].T
    in1 = in1_ref[...]

    k_t = jax.lax.broadcasted_iota(jnp.int32, (32, 256), 0)
    l_t = jax.lax.broadcasted_iota(jnp.int32, (32, 256), 1)
    same_t = (l_t // 128) == (k_t // 16)
    same_u = ((l_t % 128) // 8) == (k_t % 16)
    scale = jnp.where(k_t // 16 == 0, _C[0][0], _C[1][0]).astype(jnp.float32)
    B = jnp.where(same_t & same_u, scale, 0.0)

    r_i = jax.lax.broadcasted_iota(jnp.int32, (256, 16), 0)
    c_i = jax.lax.broadcasted_iota(jnp.int32, (256, 16), 1)
    S = jnp.where(
        ((r_i // 128) == (c_i // 8)) & ((r_i % 8) == (c_i % 8)), 1.0, 0.0
    ).astype(jnp.float32)

    in1a = in1[:, :128]
    in1b = in1[:, 128:]
    m0 = in1a + (_C[0][1] / _C[0][0]) * in1b
    m1 = in1a + (_C[1][1] / _C[1][0]) * in1b
    m = jnp.concatenate([m0, m1], axis=1)

    w = jax.lax.dot(in0, B, precision=jax.lax.Precision.DEFAULT)
    out = jax.lax.dot(w * m, S, precision=jax.lax.Precision.DEFAULT)
    outt_ref[...] = out.T


@jax.jit
def kernel(in0, in1):
    n = in0.shape[0]
    r = _BLOCK_ROWS
    grid = (pl.cdiv(n, r),)
    in0t = in0.T
    outt = pl.pallas_call(
        _body,
        grid=grid,
        in_specs=[
            pl.BlockSpec((32, r), lambda i: (0, i)),
            pl.BlockSpec((r, 256), lambda i: (i, 0)),
        ],
        out_specs=pl.BlockSpec((16, r), lambda i: (0, i)),
        out_shape=jax.ShapeDtypeStruct((16, n), in0.dtype),
        compiler_params=pltpu.CompilerParams(
            dimension_semantics=("arbitrary",),
        ),
    )(in0t, in1)
    return outt.T
